# Initial kernel scaffold; baseline (speedup 1.0000x reference)
import jax
import jax.numpy as jnp
from jax.experimental import pallas as pl


def kernel(x, edge_index, W_theta, b_theta, W_phi, b_phi):
    raise NotImplementedError("write your pallas kernel here")


# speedup vs baseline: 1.9268x; 1.9268x over previous
import functools
import jax
import jax.numpy as jnp
from jax.experimental import pallas as pl
from jax.experimental.pallas import tpu as pltpu

_N = 10000
_D = 128
_ROW_BLK = 400


def _ab_body(x_ref, wtt_ref, wst_ref, bs_ref, a_ref, b_ref):
    xb = x_ref[...]
    b_ref[...] = jnp.dot(xb, wtt_ref[...], preferred_element_type=jnp.float32)
    a_ref[...] = (
        jnp.dot(xb, wst_ref[...], preferred_element_type=jnp.float32)
        + bs_ref[...]
    )


def _compute_ab(x, W_theta, b_theta, W_phi, b_phi):
    wtt = W_theta.T
    wst = (W_theta + W_phi).T
    bs = (b_theta + b_phi).reshape(1, _D)
    grid = _N // _ROW_BLK
    a, b = pl.pallas_call(
        _ab_body,
        grid=(grid,),
        in_specs=[
            pl.BlockSpec((_ROW_BLK, _D), lambda i: (i, 0)),
            pl.BlockSpec((_D, _D), lambda i: (0, 0)),
            pl.BlockSpec((_D, _D), lambda i: (0, 0)),
            pl.BlockSpec((1, _D), lambda i: (0, 0)),
        ],
        out_specs=[
            pl.BlockSpec((_ROW_BLK, _D), lambda i: (i, 0)),
            pl.BlockSpec((_ROW_BLK, _D), lambda i: (i, 0)),
        ],
        out_shape=[
            jax.ShapeDtypeStruct((_N, _D), jnp.float32),
            jax.ShapeDtypeStruct((_N, _D), jnp.float32),
        ],
    )(x, wtt, wst, bs)
    return a, b


def kernel(x, edge_index, W_theta, b_theta, W_phi, b_phi):
    a, b = _compute_ab(x, W_theta, b_theta, W_phi, b_phi)
    src = edge_index[0]
    dst = edge_index[1]
    m = jax.ops.segment_min(b[src], dst, num_segments=_N)
    return jnp.where(jnp.isfinite(m), a - m, 0.0)
